# baseline (speedup 1.0000x reference)
import functools

import jax
import jax.numpy as jnp
from jax import lax
from jax.experimental import pallas as pl
from jax.experimental.pallas import tpu as pltpu
from jax.experimental.pallas import tpu_sc as plsc

B = 16384
W_DIM, C_DIM, SC_DIM, I_DIM = 64, 16, 32, 16
N_W, N_P, N_C = 16, 16, 8
FC_IN = W_DIM + C_DIM + SC_DIM + I_DIM + N_W + N_P + N_C
FC1, FC2 = 128, 32


def _make_sc_gather():
    info = plsc.get_sparse_core_info()
    nw = info.num_cores * info.num_subcores
    b_per_w = B // nw
    mesh = plsc.VectorSubcoreMesh(core_axis_name="c", subcore_axis_name="s")

    @functools.partial(
        pl.kernel,
        mesh=mesh,
        compiler_params=pltpu.CompilerParams(use_tc_tiling_on_sc=False),
        out_type=[
            jax.ShapeDtypeStruct((B, W_DIM), jnp.float32),
            jax.ShapeDtypeStruct((B, C_DIM), jnp.float32),
            jax.ShapeDtypeStruct((B, SC_DIM), jnp.float32),
            jax.ShapeDtypeStruct((B, I_DIM), jnp.float32),
        ],
        scratch_types=[
            pltpu.VMEM((b_per_w,), jnp.int32),
            pltpu.VMEM((b_per_w,), jnp.int32),
            pltpu.VMEM((b_per_w,), jnp.int32),
            pltpu.VMEM((b_per_w,), jnp.int32),
            pltpu.VMEM((b_per_w, W_DIM), jnp.float32),
            pltpu.VMEM((b_per_w, C_DIM), jnp.float32),
            pltpu.VMEM((b_per_w, SC_DIM), jnp.float32),
            pltpu.VMEM((b_per_w, I_DIM), jnp.float32),
            pltpu.SemaphoreType.DMA,
        ],
    )
    def sc_gather(wid_hbm, cid_hbm, sid_hbm, iid_hbm,
                  ww_hbm, wc_hbm, ws_hbm, wi_hbm,
                  ow_hbm, oc_hbm, os_hbm, oi_hbm,
                  wi_v, ci_v, si_v, ii_v,
                  wr_v, cr_v, sr_v, ir_v, sem):
        w = lax.axis_index("s") * info.num_cores + lax.axis_index("c")
        base = w * b_per_w
        pltpu.sync_copy(wid_hbm.at[pl.ds(base, b_per_w)], wi_v)
        pltpu.sync_copy(cid_hbm.at[pl.ds(base, b_per_w)], ci_v)
        pltpu.sync_copy(sid_hbm.at[pl.ds(base, b_per_w)], si_v)
        pltpu.sync_copy(iid_hbm.at[pl.ds(base, b_per_w)], ii_v)
        c0 = pltpu.async_copy(ww_hbm.at[wi_v], wr_v, sem)
        c1 = pltpu.async_copy(wc_hbm.at[ci_v], cr_v, sem)
        c2 = pltpu.async_copy(ws_hbm.at[si_v], sr_v, sem)
        c3 = pltpu.async_copy(wi_hbm.at[ii_v], ir_v, sem)
        c0.wait()
        c1.wait()
        c2.wait()
        c3.wait()
        pltpu.sync_copy(wr_v, ow_hbm.at[pl.ds(base, b_per_w)])
        pltpu.sync_copy(cr_v, oc_hbm.at[pl.ds(base, b_per_w)])
        pltpu.sync_copy(sr_v, os_hbm.at[pl.ds(base, b_per_w)])
        pltpu.sync_copy(ir_v, oi_hbm.at[pl.ds(base, b_per_w)])

    return sc_gather


_sc_gather = None


def _mlp_body(we_ref, ce_ref, se_ref, ie_ref, nw_ref, np_ref, nc_ref,
              w1_ref, b1_ref, w2_ref, b2_ref, w3t_ref, b3_ref, out_ref):
    feats = jnp.concatenate(
        [we_ref[...], ce_ref[...], se_ref[...], ie_ref[...],
         nw_ref[...], np_ref[...], nc_ref[...]], axis=1)
    x = jnp.dot(feats, w1_ref[...], preferred_element_type=jnp.float32)
    x = jnp.maximum(x + b1_ref[...], 0.0)
    x = jnp.dot(x, w2_ref[...], preferred_element_type=jnp.float32)
    x = jnp.maximum(x + b2_ref[...], 0.0)
    out_ref[...] = jnp.sum(x * w3t_ref[...], axis=1, keepdims=True) + b3_ref[...]


def _mlp(we, ce, se, ie, nw, npf, ncf, w1, b1, w2, b2, w3, b3, bt=2048):
    grid = B // bt
    ds = lambda i: (i, 0)
    ws = lambda i: (0, 0)
    return pl.pallas_call(
        _mlp_body,
        grid=(grid,),
        in_specs=[
            pl.BlockSpec((bt, W_DIM), ds),
            pl.BlockSpec((bt, C_DIM), ds),
            pl.BlockSpec((bt, SC_DIM), ds),
            pl.BlockSpec((bt, I_DIM), ds),
            pl.BlockSpec((bt, N_W), ds),
            pl.BlockSpec((bt, N_P), ds),
            pl.BlockSpec((bt, N_C), ds),
            pl.BlockSpec((FC_IN, FC1), ws),
            pl.BlockSpec((1, FC1), ws),
            pl.BlockSpec((FC1, FC2), ws),
            pl.BlockSpec((1, FC2), ws),
            pl.BlockSpec((1, FC2), ws),
            pl.BlockSpec((1, 1), ws),
        ],
        out_specs=pl.BlockSpec((bt, 1), ds),
        out_shape=jax.ShapeDtypeStruct((B, 1), jnp.float32),
    )(we, ce, se, ie, nw, npf, ncf,
      w1, b1.reshape(1, FC1), w2, b2.reshape(1, FC2),
      w3.reshape(1, FC2), b3.reshape(1, 1))


def kernel(worker_ids, cat_ids, sub_cat_ids, ind_ids,
           numeric_worker_feats, numeric_project_feats, numeric_context_feats,
           W_worker, W_cat, W_sub, W_ind, W1, b1, W2, b2, W3, b3):
    global _sc_gather
    if _sc_gather is None:
        _sc_gather = _make_sc_gather()
    we, ce, se, ie = _sc_gather(
        worker_ids.astype(jnp.int32), cat_ids.astype(jnp.int32),
        sub_cat_ids.astype(jnp.int32), ind_ids.astype(jnp.int32),
        W_worker, W_cat, W_sub, W_ind)
    return _mlp(we, ce, se, ie,
                numeric_worker_feats, numeric_project_feats,
                numeric_context_feats, W1, b1, W2, b2, W3, b3)


# speedup vs baseline: 2.4798x; 2.4798x over previous
import functools

import jax
import jax.numpy as jnp
from jax import lax
from jax.experimental import pallas as pl
from jax.experimental.pallas import tpu as pltpu
from jax.experimental.pallas import tpu_sc as plsc

B = 16384
W_DIM, C_DIM, SC_DIM, I_DIM = 64, 16, 32, 16
N_W, N_P, N_C = 16, 16, 8
EMB = W_DIM + C_DIM + SC_DIM + I_DIM
FC_IN = EMB + N_W + N_P + N_C
FC1, FC2 = 128, 32
NUM_WORKERS, NUM_CATS, NUM_SUBCATS, NUM_INDS = 1000000, 1000, 100000, 1000

R = 8
OFF_W, OFF_C, OFF_S, OFF_I = 0, W_DIM, W_DIM + C_DIM, W_DIM + C_DIM + SC_DIM


def _make_sc_gather():
    info = plsc.get_sparse_core_info()
    nw = info.num_cores * info.num_subcores
    b_per_w = B // nw
    mesh = plsc.VectorSubcoreMesh(core_axis_name="c", subcore_axis_name="s")

    @functools.partial(
        pl.kernel,
        mesh=mesh,
        out_type=jax.ShapeDtypeStruct((B, EMB), jnp.float32),
        scratch_types=[
            pltpu.VMEM((b_per_w,), jnp.int32),
            pltpu.VMEM((b_per_w,), jnp.int32),
            pltpu.VMEM((b_per_w,), jnp.int32),
            pltpu.VMEM((b_per_w,), jnp.int32),
            pltpu.VMEM((b_per_w, EMB), jnp.float32),
            pltpu.SemaphoreType.DMA,
        ],
    )
    def sc_gather(wid_hbm, cid_hbm, sid_hbm, iid_hbm,
                  ww_hbm, wc_hbm, ws_hbm, wi_hbm,
                  out_hbm,
                  wi_v, ci_v, si_v, ii_v, ob_v, sem):
        w = lax.axis_index("s") * info.num_cores + lax.axis_index("c")
        base = w * b_per_w
        pltpu.sync_copy(wid_hbm.at[pl.ds(base, b_per_w)], wi_v)
        pltpu.sync_copy(cid_hbm.at[pl.ds(base, b_per_w)], ci_v)
        pltpu.sync_copy(sid_hbm.at[pl.ds(base, b_per_w)], si_v)
        pltpu.sync_copy(iid_hbm.at[pl.ds(base, b_per_w)], ii_v)

        @pl.loop(0, b_per_w // 16)
        def _grp(g):
            gbase = g * 16
            wv = wi_v[pl.ds(gbase, 16)]
            cv = ci_v[pl.ds(gbase, 16)]
            sv = si_v[pl.ds(gbase, 16)]
            iv = ii_v[pl.ds(gbase, 16)]
            for j in range(16):
                i = gbase + j
                wid, cid, sid, iid = wv[j], cv[j], sv[j], iv[j]
                pltpu.async_copy(ww_hbm.at[wid >> 3, wid & 7],
                                 ob_v.at[i, pl.ds(OFF_W, W_DIM)], sem)
                pltpu.async_copy(wc_hbm.at[cid >> 3, cid & 7],
                                 ob_v.at[i, pl.ds(OFF_C, C_DIM)], sem)
                pltpu.async_copy(ws_hbm.at[sid >> 3, sid & 7],
                                 ob_v.at[i, pl.ds(OFF_S, SC_DIM)], sem)
                pltpu.async_copy(wi_hbm.at[iid >> 3, iid & 7],
                                 ob_v.at[i, pl.ds(OFF_I, I_DIM)], sem)

        pltpu.make_async_copy(
            out_hbm.at[pl.ds(0, b_per_w)], ob_v, sem).wait()
        pltpu.sync_copy(ob_v, out_hbm.at[pl.ds(base, b_per_w)])

    return sc_gather


_sc_gather = None


def _mlp_body(emb_ref, nw_ref, np_ref, nc_ref,
              w1_ref, b1_ref, w2_ref, b2_ref, w3t_ref, b3_ref, out_ref):
    feats = jnp.concatenate(
        [emb_ref[...], nw_ref[...], np_ref[...], nc_ref[...]], axis=1)
    x = jnp.dot(feats, w1_ref[...], preferred_element_type=jnp.float32)
    x = jnp.maximum(x + b1_ref[...], 0.0)
    x = jnp.dot(x, w2_ref[...], preferred_element_type=jnp.float32)
    x = jnp.maximum(x + b2_ref[...], 0.0)
    out_ref[...] = jnp.sum(x * w3t_ref[...], axis=1, keepdims=True) + b3_ref[...]


def _mlp(emb, nw, npf, ncf, w1, b1, w2, b2, w3, b3, bt=2048):
    grid = B // bt
    ds = lambda i: (i, 0)
    ws = lambda i: (0, 0)
    return pl.pallas_call(
        _mlp_body,
        grid=(grid,),
        in_specs=[
            pl.BlockSpec((bt, EMB), ds),
            pl.BlockSpec((bt, N_W), ds),
            pl.BlockSpec((bt, N_P), ds),
            pl.BlockSpec((bt, N_C), ds),
            pl.BlockSpec((FC_IN, FC1), ws),
            pl.BlockSpec((1, FC1), ws),
            pl.BlockSpec((FC1, FC2), ws),
            pl.BlockSpec((1, FC2), ws),
            pl.BlockSpec((1, FC2), ws),
            pl.BlockSpec((1, 1), ws),
        ],
        out_specs=pl.BlockSpec((bt, 1), ds),
        out_shape=jax.ShapeDtypeStruct((B, 1), jnp.float32),
    )(emb, nw, npf, ncf,
      w1, b1.reshape(1, FC1), w2, b2.reshape(1, FC2),
      w3.reshape(1, FC2), b3.reshape(1, 1))


def kernel(worker_ids, cat_ids, sub_cat_ids, ind_ids,
           numeric_worker_feats, numeric_project_feats, numeric_context_feats,
           W_worker, W_cat, W_sub, W_ind, W1, b1, W2, b2, W3, b3):
    global _sc_gather
    if _sc_gather is None:
        _sc_gather = _make_sc_gather()
    emb = _sc_gather(
        worker_ids.astype(jnp.int32), cat_ids.astype(jnp.int32),
        sub_cat_ids.astype(jnp.int32), ind_ids.astype(jnp.int32),
        W_worker.reshape(NUM_WORKERS // R, R, W_DIM),
        W_cat.reshape(NUM_CATS // R, R, C_DIM),
        W_sub.reshape(NUM_SUBCATS // R, R, SC_DIM),
        W_ind.reshape(NUM_INDS // R, R, I_DIM))
    return _mlp(emb, numeric_worker_feats, numeric_project_feats,
                numeric_context_feats, W1, b1, W2, b2, W3, b3)


# speedup vs baseline: 2.4891x; 1.0038x over previous
import functools

import jax
import jax.numpy as jnp
from jax import lax
from jax.experimental import pallas as pl
from jax.experimental.pallas import tpu as pltpu
from jax.experimental.pallas import tpu_sc as plsc

B = 16384
W_DIM, C_DIM, SC_DIM, I_DIM = 64, 16, 32, 16
N_W, N_P, N_C = 16, 16, 8
EMB = W_DIM + C_DIM + SC_DIM + I_DIM
FC_IN = EMB + N_W + N_P + N_C
FC1, FC2 = 128, 32
NUM_WORKERS, NUM_CATS, NUM_SUBCATS, NUM_INDS = 1000000, 1000, 100000, 1000

R_BIG = 16
R_SMALL = 8
OFF_W, OFF_C, OFF_S, OFF_I = 0, W_DIM, W_DIM + C_DIM, W_DIM + C_DIM + SC_DIM


def _make_sc_gather():
    info = plsc.get_sparse_core_info()
    nw = info.num_cores * info.num_subcores
    b_per_w = B // nw
    mesh = plsc.VectorSubcoreMesh(core_axis_name="c", subcore_axis_name="s")

    @functools.partial(
        pl.kernel,
        mesh=mesh,
        out_type=jax.ShapeDtypeStruct((B, EMB), jnp.float32),
        scratch_types=[
            pltpu.VMEM((b_per_w,), jnp.int32),
            pltpu.VMEM((b_per_w,), jnp.int32),
            pltpu.VMEM((b_per_w,), jnp.int32),
            pltpu.VMEM((b_per_w,), jnp.int32),
            pltpu.VMEM((b_per_w, EMB), jnp.float32),
            pltpu.SemaphoreType.DMA,
        ],
    )
    def sc_gather(wid_hbm, cid_hbm, sid_hbm, iid_hbm,
                  ww_hbm, wc_hbm, ws_hbm, wi_hbm,
                  out_hbm,
                  wi_v, ci_v, si_v, ii_v, ob_v, sem):
        w = lax.axis_index("s") * info.num_cores + lax.axis_index("c")
        base = w * b_per_w
        pltpu.sync_copy(wid_hbm.at[pl.ds(base, b_per_w)], wi_v)
        pltpu.sync_copy(cid_hbm.at[pl.ds(base, b_per_w)], ci_v)
        pltpu.sync_copy(sid_hbm.at[pl.ds(base, b_per_w)], si_v)
        pltpu.sync_copy(iid_hbm.at[pl.ds(base, b_per_w)], ii_v)

        @pl.loop(0, b_per_w // 16)
        def _grp(g):
            gbase = g * 16
            wv = wi_v[pl.ds(gbase, 16)]
            cv = ci_v[pl.ds(gbase, 16)]
            sv = si_v[pl.ds(gbase, 16)]
            iv = ii_v[pl.ds(gbase, 16)]
            for j in range(16):
                i = gbase + j
                wid, cid, sid, iid = wv[j], cv[j], sv[j], iv[j]
                pltpu.async_copy(ww_hbm.at[wid >> 4, wid & 15],
                                 ob_v.at[i, pl.ds(OFF_W, W_DIM)], sem)
                pltpu.async_copy(wc_hbm.at[cid >> 3, cid & 7],
                                 ob_v.at[i, pl.ds(OFF_C, C_DIM)], sem)
                pltpu.async_copy(ws_hbm.at[sid >> 4, sid & 15],
                                 ob_v.at[i, pl.ds(OFF_S, SC_DIM)], sem)
                pltpu.async_copy(wi_hbm.at[iid >> 3, iid & 7],
                                 ob_v.at[i, pl.ds(OFF_I, I_DIM)], sem)

        pltpu.make_async_copy(
            out_hbm.at[pl.ds(0, b_per_w)], ob_v, sem).wait()
        pltpu.sync_copy(ob_v, out_hbm.at[pl.ds(base, b_per_w)])

    return sc_gather


_sc_gather = None


def _mlp_body(emb_ref, nw_ref, np_ref, nc_ref,
              w1_ref, b1_ref, w2_ref, b2_ref, w3t_ref, b3_ref, out_ref):
    feats = jnp.concatenate(
        [emb_ref[...], nw_ref[...], np_ref[...], nc_ref[...]], axis=1)
    x = jnp.dot(feats, w1_ref[...], preferred_element_type=jnp.float32)
    x = jnp.maximum(x + b1_ref[...], 0.0)
    x = jnp.dot(x, w2_ref[...], preferred_element_type=jnp.float32)
    x = jnp.maximum(x + b2_ref[...], 0.0)
    out_ref[...] = jnp.sum(x * w3t_ref[...], axis=1, keepdims=True) + b3_ref[...]


def _mlp(emb, nw, npf, ncf, w1, b1, w2, b2, w3, b3, bt=2048):
    grid = B // bt
    ds = lambda i: (i, 0)
    ws = lambda i: (0, 0)
    return pl.pallas_call(
        _mlp_body,
        grid=(grid,),
        in_specs=[
            pl.BlockSpec((bt, EMB), ds),
            pl.BlockSpec((bt, N_W), ds),
            pl.BlockSpec((bt, N_P), ds),
            pl.BlockSpec((bt, N_C), ds),
            pl.BlockSpec((FC_IN, FC1), ws),
            pl.BlockSpec((1, FC1), ws),
            pl.BlockSpec((FC1, FC2), ws),
            pl.BlockSpec((1, FC2), ws),
            pl.BlockSpec((1, FC2), ws),
            pl.BlockSpec((1, 1), ws),
        ],
        out_specs=pl.BlockSpec((bt, 1), ds),
        out_shape=jax.ShapeDtypeStruct((B, 1), jnp.float32),
    )(emb, nw, npf, ncf,
      w1, b1.reshape(1, FC1), w2, b2.reshape(1, FC2),
      w3.reshape(1, FC2), b3.reshape(1, 1))


def kernel(worker_ids, cat_ids, sub_cat_ids, ind_ids,
           numeric_worker_feats, numeric_project_feats, numeric_context_feats,
           W_worker, W_cat, W_sub, W_ind, W1, b1, W2, b2, W3, b3):
    global _sc_gather
    if _sc_gather is None:
        _sc_gather = _make_sc_gather()
    emb = _sc_gather(
        worker_ids.astype(jnp.int32), cat_ids.astype(jnp.int32),
        sub_cat_ids.astype(jnp.int32), ind_ids.astype(jnp.int32),
        W_worker.reshape(NUM_WORKERS // R_BIG, R_BIG, W_DIM),
        W_cat.reshape(NUM_CATS // R_SMALL, R_SMALL, C_DIM),
        W_sub.reshape(NUM_SUBCATS // R_BIG, R_BIG, SC_DIM),
        W_ind.reshape(NUM_INDS // R_SMALL, R_SMALL, I_DIM))
    return _mlp(emb, numeric_worker_feats, numeric_project_feats,
                numeric_context_feats, W1, b1, W2, b2, W3, b3)
